# baseline (speedup 1.0000x reference)
import functools

import jax
import jax.numpy as jnp
import numpy as np
from jax import lax
from jax.experimental import pallas as pl
from jax.experimental.pallas import tpu as pltpu
from jax.experimental.pallas import tpu_sc as plsc

_N_A = 512
_N_OBS = 16
_K = 5
_N = 2 * _N_A + _N_OBS
_E = _N * _K + _N_OBS

_INTERPRET = False


def _knn_body(posT_ref, pos_ref, dists_ref, srcs_ref, poss_ref, agp_ref,
              cnt_ref, d2_ref):
    x_row = posT_ref[0, 0:1, :]
    y_row = posT_ref[0, 1:2, :]
    x_col = pos_ref[0, :, 0:1]
    y_col = pos_ref[0, :, 1:2]
    sq_row = x_row * x_row + y_row * y_row
    sq_col = x_col * x_col + y_col * y_col
    dot = lax.dot(pos_ref[0].astype(jnp.bfloat16),
                  posT_ref[0].astype(jnp.bfloat16),
                  preferred_element_type=jnp.float32)
    d2 = (sq_col + sq_row) - 2.0 * dot
    d2 = jnp.maximum(d2, 0.0)
    rows = lax.broadcasted_iota(jnp.int32, (_N, _N), 0)
    cols = lax.broadcasted_iota(jnp.int32, (_N, _N), 1)
    inf = jnp.float32(jnp.inf)
    d2 = jnp.where(rows == cols, inf, d2)
    d2_ref[...] = d2

    for k in range(_K):
        d2c = d2_ref[...]
        m = jnp.min(d2c, axis=1, keepdims=True)
        am = jnp.min(jnp.where(d2c <= m, cols, _N), axis=1, keepdims=True)
        d2_ref[...] = jnp.where(cols == am, inf, d2c)
        dists_ref[0, :, k : k + 1] = jnp.sqrt(jnp.maximum(m, 1e-12))
        srcs_ref[0, :, k : k + 1] = am

    d2c = d2_ref[...]
    qmask = ((d2c == inf) & (rows != cols)) | (
        (rows == cols + _N_A) & (cols < _N_OBS))
    q = jnp.where(qmask, 1.0, 0.0).astype(jnp.float32)
    lf = jnp.where(rows > cols, 1.0, 0.0).astype(jnp.float32)
    pc = lax.dot(lf.astype(jnp.bfloat16), q.astype(jnp.bfloat16),
                 preferred_element_type=jnp.float32)
    colcount = pc[_N - 1 : _N, :] + q[_N - 1 : _N, :]
    colstart = lax.dot_general(
        colcount, lf, (((1,), (1,)), ((), ())),
        precision=lax.Precision.HIGHEST,
        preferred_element_type=jnp.float32)
    r = pc + colstart
    d2_ref[...] = r
    cnt_ref[0, pl.program_id(0)] = jnp.sum(colcount).astype(jnp.int32)

    for k in range(_K):
        am = srcs_ref[0, :, k : k + 1]
        pos_k = jnp.sum(jnp.where(cols == am, r, 0.0), axis=1, keepdims=True)
        poss_ref[0, :, k : k + 1] = pos_k.astype(jnp.int32)
    rsl = r[_N_A : _N_A + _N_OBS, :]
    m16 = (lax.broadcasted_iota(jnp.int32, (_N_OBS, _N), 0)
           == lax.broadcasted_iota(jnp.int32, (_N_OBS, _N), 1))
    agp = jnp.sum(jnp.where(m16, rsl, 0.0), axis=1, keepdims=True)
    agp_ref[0, :, :] = agp.astype(jnp.int32)


def _knn_pallas(posT, pos):
    Bv = pos.shape[0]
    return pl.pallas_call(
        _knn_body,
        grid=(Bv,),
        in_specs=[
            pl.BlockSpec((1, 2, _N), lambda b: (b, 0, 0)),
            pl.BlockSpec((1, _N, 2), lambda b: (b, 0, 0)),
        ],
        out_specs=[
            pl.BlockSpec((1, _N, _K), lambda b: (b, 0, 0)),
            pl.BlockSpec((1, _N, _K), lambda b: (b, 0, 0)),
            pl.BlockSpec((1, _N, _K), lambda b: (b, 0, 0)),
            pl.BlockSpec((1, _N_OBS, 1), lambda b: (b, 0, 0)),
            pl.BlockSpec((1, Bv), lambda b: (0, 0), memory_space=pltpu.SMEM),
        ],
        out_shape=[
            jax.ShapeDtypeStruct((Bv, _N, _K), jnp.float32),
            jax.ShapeDtypeStruct((Bv, _N, _K), jnp.int32),
            jax.ShapeDtypeStruct((Bv, _N, _K), jnp.int32),
            jax.ShapeDtypeStruct((Bv, _N_OBS, 1), jnp.int32),
            jax.ShapeDtypeStruct((1, Bv), jnp.int32),
        ],
        scratch_shapes=[
            pltpu.VMEM((_N, _N), jnp.float32),
        ],
        compiler_params=pltpu.CompilerParams(
            dimension_semantics=("parallel",)),
        interpret=_INTERPRET,
    )(posT, pos)


def _scatter_sc(pos_all, src_all, ctr):
    Bv = pos_all.shape[0]
    n_workers = 32
    bpw = Bv // n_workers
    n_vregs = _E // 16

    @functools.partial(
        pl.kernel,
        mesh=plsc.VectorSubcoreMesh(core_axis_name="c", subcore_axis_name="s"),
        compiler_params=pltpu.CompilerParams(needs_layout_passes=False),
        out_type=jax.ShapeDtypeStruct((Bv, 2, _E), jnp.int32),
        scratch_types=[
            pltpu.VMEM((_E,), jnp.int32),
            pltpu.VMEM((_E,), jnp.int32),
            pltpu.VMEM((_E,), jnp.int32),
            pltpu.VMEM((_E,), jnp.int32),
            pltpu.VMEM((_E,), jnp.int32),
        ],
    )
    def scatter_kernel(pos_hbm, src_hbm, ctr_hbm, out_hbm,
                       pos_v, src_v, ctr_v, out0_v, out1_v):
        wid = lax.axis_index("s") * 2 + lax.axis_index("c")
        pltpu.sync_copy(ctr_hbm, ctr_v)
        neg1 = jnp.full((16,), -1, jnp.int32)
        for j in range(bpw):
            b = wid * bpw + j
            pltpu.sync_copy(pos_hbm.at[b], pos_v)
            pltpu.sync_copy(src_hbm.at[b], src_v)

            def init_body(i, carry):
                out0_v[pl.ds(i * 16, 16)] = neg1
                out1_v[pl.ds(i * 16, 16)] = neg1
                return carry

            lax.fori_loop(0, n_vregs, init_body, 0)

            def scat_body(i, carry):
                p = pos_v[pl.ds(i * 16, 16)]
                s = src_v[pl.ds(i * 16, 16)]
                c = ctr_v[pl.ds(i * 16, 16)]
                plsc.store_scatter(out0_v, [p], s)
                plsc.store_scatter(out1_v, [p], c)
                return carry

            lax.fori_loop(0, n_vregs, scat_body, 0)
            pltpu.sync_copy(out0_v, out_hbm.at[b, 0])
            pltpu.sync_copy(out1_v, out_hbm.at[b, 1])

    return scatter_kernel(pos_all, src_all, ctr)


def kernel(obs, state):
    Bv = obs.shape[0]
    agent_pos = obs[:, :, 0:2]
    goal_pos = obs[:, :, 4:6]
    pos = jnp.concatenate([agent_pos, goal_pos, state], axis=1)
    posT = pos.transpose(0, 2, 1)

    dists, srcs, poss, ag_pos, counts = _knn_pallas(posT, pos)

    pos_all = jnp.concatenate(
        [poss.reshape(Bv, _N * _K), ag_pos.reshape(Bv, _N_OBS)], axis=1)
    src_all = jnp.concatenate(
        [srcs.reshape(Bv, _N * _K),
         jnp.broadcast_to(jnp.arange(_N_OBS, dtype=jnp.int32)[None, :],
                          (Bv, _N_OBS))], axis=1)
    ctr_np = np.concatenate(
        [np.repeat(np.arange(_N), _K), _N_A + np.arange(_N_OBS)]
    ).astype(np.int32)
    out_edges = _scatter_sc(pos_all, src_all, jnp.asarray(ctr_np))

    x = np.zeros((_N, 4), dtype=np.float32)
    x[:_N_OBS, 0] = 1.0
    x[_N_OBS : _N_OBS + _N_A, 1] = 1.0
    x[_N_OBS + _N_A :, 2] = 1.0
    x_all = jnp.broadcast_to(jnp.asarray(x)[None, :, :], (Bv, _N, 4))
    return (x_all, out_edges, counts.reshape(Bv), dists)


# speedup vs baseline: 1.0016x; 1.0016x over previous
import functools

import jax
import jax.numpy as jnp
import numpy as np
from jax import lax
from jax.experimental import pallas as pl
from jax.experimental.pallas import tpu as pltpu
from jax.experimental.pallas import tpu_sc as plsc

_N_A = 512
_N_OBS = 16
_K = 5
_N = 2 * _N_A + _N_OBS
_E = _N * _K + _N_OBS


def _knn_body(posT_ref, pos_ref, dists_ref, srcs_ref, poss_ref, agp_ref,
              cnt_ref, d2_ref):
    x_row = posT_ref[0, 0:1, :]
    y_row = posT_ref[0, 1:2, :]
    x_col = pos_ref[0, :, 0:1]
    y_col = pos_ref[0, :, 1:2]
    sq_row = x_row * x_row + y_row * y_row
    sq_col = x_col * x_col + y_col * y_col
    dot = lax.dot(pos_ref[0].astype(jnp.bfloat16),
                  posT_ref[0].astype(jnp.bfloat16),
                  preferred_element_type=jnp.float32)
    d2 = (sq_col + sq_row) - 2.0 * dot
    d2 = jnp.maximum(d2, 0.0)
    rows = lax.broadcasted_iota(jnp.int32, (_N, _N), 0)
    cols = lax.broadcasted_iota(jnp.int32, (_N, _N), 1)
    inf = jnp.float32(jnp.inf)
    d2 = jnp.where(rows == cols, inf, d2)
    d2_ref[...] = d2

    for k in range(_K):
        d2c = d2_ref[...]
        m = jnp.min(d2c, axis=1, keepdims=True)
        am = jnp.min(jnp.where(d2c <= m, cols, _N), axis=1, keepdims=True)
        d2_ref[...] = jnp.where(cols == am, inf, d2c)
        dists_ref[0, :, k : k + 1] = jnp.sqrt(jnp.maximum(m, 1e-12))
        srcs_ref[0, :, k : k + 1] = am

    d2c = d2_ref[...]
    qmask = ((d2c == inf) & (rows != cols)) | (
        (rows == cols + _N_A) & (cols < _N_OBS))
    q = jnp.where(qmask, 1.0, 0.0).astype(jnp.float32)
    lf = jnp.where(rows > cols, 1.0, 0.0).astype(jnp.float32)
    pc = lax.dot(lf.astype(jnp.bfloat16), q.astype(jnp.bfloat16),
                 preferred_element_type=jnp.float32)
    colcount = pc[_N - 1 : _N, :] + q[_N - 1 : _N, :]
    colstart = lax.dot_general(
        colcount, lf, (((1,), (1,)), ((), ())),
        precision=lax.Precision.HIGHEST,
        preferred_element_type=jnp.float32)
    r = pc + colstart
    d2_ref[...] = r
    cnt_ref[0, pl.program_id(0)] = jnp.sum(colcount).astype(jnp.int32)

    for k in range(_K):
        am = srcs_ref[0, :, k : k + 1]
        pos_k = jnp.sum(jnp.where(cols == am, r, 0.0), axis=1, keepdims=True)
        poss_ref[0, :, k : k + 1] = pos_k.astype(jnp.int32)
    rsl = r[_N_A : _N_A + _N_OBS, :]
    m16 = (lax.broadcasted_iota(jnp.int32, (_N_OBS, _N), 0)
           == lax.broadcasted_iota(jnp.int32, (_N_OBS, _N), 1))
    agp = jnp.sum(jnp.where(m16, rsl, 0.0), axis=1, keepdims=True)
    agp_ref[0, :, :] = agp.astype(jnp.int32)


def _knn_pallas(posT, pos):
    Bv = pos.shape[0]
    return pl.pallas_call(
        _knn_body,
        grid=(Bv,),
        in_specs=[
            pl.BlockSpec((1, 2, _N), lambda b: (b, 0, 0)),
            pl.BlockSpec((1, _N, 2), lambda b: (b, 0, 0)),
        ],
        out_specs=[
            pl.BlockSpec((1, _N, _K), lambda b: (b, 0, 0)),
            pl.BlockSpec((1, _N, _K), lambda b: (b, 0, 0)),
            pl.BlockSpec((1, _N, _K), lambda b: (b, 0, 0)),
            pl.BlockSpec((1, _N_OBS, 1), lambda b: (b, 0, 0)),
            pl.BlockSpec((1, Bv), lambda b: (0, 0), memory_space=pltpu.SMEM),
        ],
        out_shape=[
            jax.ShapeDtypeStruct((Bv, _N, _K), jnp.float32),
            jax.ShapeDtypeStruct((Bv, _N, _K), jnp.int32),
            jax.ShapeDtypeStruct((Bv, _N, _K), jnp.int32),
            jax.ShapeDtypeStruct((Bv, _N_OBS, 1), jnp.int32),
            jax.ShapeDtypeStruct((1, Bv), jnp.int32),
        ],
        scratch_shapes=[
            pltpu.VMEM((_N, _N), jnp.float32),
        ],
    )(posT, pos)


def _scatter_sc(pos_all, src_all, ctr):
    Bv = pos_all.shape[0]
    n_workers = 32
    bpw = Bv // n_workers
    n_vregs = _E // 16

    @functools.partial(
        pl.kernel,
        mesh=plsc.VectorSubcoreMesh(core_axis_name="c", subcore_axis_name="s"),
        compiler_params=pltpu.CompilerParams(needs_layout_passes=False),
        out_type=jax.ShapeDtypeStruct((Bv, 2, _E), jnp.int32),
        scratch_types=[
            pltpu.VMEM((_E,), jnp.int32),
            pltpu.VMEM((_E,), jnp.int32),
            pltpu.VMEM((_E,), jnp.int32),
            pltpu.VMEM((_E,), jnp.int32),
            pltpu.VMEM((_E,), jnp.int32),
        ],
    )
    def scatter_kernel(pos_hbm, src_hbm, ctr_hbm, out_hbm,
                       pos_v, src_v, ctr_v, out0_v, out1_v):
        wid = lax.axis_index("s") * 2 + lax.axis_index("c")
        pltpu.sync_copy(ctr_hbm, ctr_v)
        neg1 = jnp.full((16,), -1, jnp.int32)
        for j in range(bpw):
            b = wid * bpw + j
            pltpu.sync_copy(pos_hbm.at[b], pos_v)
            pltpu.sync_copy(src_hbm.at[b], src_v)

            def init_body(i, carry):
                out0_v[pl.ds(i * 16, 16)] = neg1
                out1_v[pl.ds(i * 16, 16)] = neg1
                return carry

            lax.fori_loop(0, n_vregs, init_body, 0)

            def scat_body(i, carry):
                p = pos_v[pl.ds(i * 16, 16)]
                s = src_v[pl.ds(i * 16, 16)]
                c = ctr_v[pl.ds(i * 16, 16)]
                plsc.store_scatter(out0_v, [p], s)
                plsc.store_scatter(out1_v, [p], c)
                return carry

            lax.fori_loop(0, n_vregs, scat_body, 0)
            pltpu.sync_copy(out0_v, out_hbm.at[b, 0])
            pltpu.sync_copy(out1_v, out_hbm.at[b, 1])

    return scatter_kernel(pos_all, src_all, ctr)


def kernel(obs, state):
    Bv = obs.shape[0]
    agent_pos = obs[:, :, 0:2]
    goal_pos = obs[:, :, 4:6]
    pos = jnp.concatenate([agent_pos, goal_pos, state], axis=1)
    posT = pos.transpose(0, 2, 1)

    dists, srcs, poss, ag_pos, counts = _knn_pallas(posT, pos)

    pos_all = jnp.concatenate(
        [poss.reshape(Bv, _N * _K), ag_pos.reshape(Bv, _N_OBS)], axis=1)
    src_all = jnp.concatenate(
        [srcs.reshape(Bv, _N * _K),
         jnp.broadcast_to(jnp.arange(_N_OBS, dtype=jnp.int32)[None, :],
                          (Bv, _N_OBS))], axis=1)
    ctr_np = np.concatenate(
        [np.repeat(np.arange(_N), _K), _N_A + np.arange(_N_OBS)]
    ).astype(np.int32)
    out_edges = _scatter_sc(pos_all, src_all, jnp.asarray(ctr_np))

    x = np.zeros((_N, 4), dtype=np.float32)
    x[:_N_OBS, 0] = 1.0
    x[_N_OBS : _N_OBS + _N_A, 1] = 1.0
    x[_N_OBS + _N_A :, 2] = 1.0
    x_all = jnp.broadcast_to(jnp.asarray(x)[None, :, :], (Bv, _N, 4))
    return (x_all, out_edges, counts.reshape(Bv), dists)
